# baseline (speedup 1.0000x reference)
import jax
import jax.numpy as jnp
from jax import lax
from jax.experimental import pallas as pl
from jax.experimental.pallas import tpu as pltpu

B = 16
N = 16
D = 128
SEQ = 2048
KW = 2 * N * N
BN = B * N
GRP = 16
NG = SEQ // GRP
G = 64
KV_ROWS = BN * SEQ // G
KW_ROWS = B * SEQ // G


def _copy_kernel(k_in, v_in, kw_in, k_out, v_out, kw_out):
    k_out[...] = k_in[...]
    v_out[...] = v_in[...]
    kw_out[...] = kw_in[...]


_COPY_SPEC = dict(
    grid=(G,),
    in_specs=[
        pl.BlockSpec((KV_ROWS, D), lambda i: (i, 0)),
        pl.BlockSpec((KV_ROWS, D), lambda i: (i, 0)),
        pl.BlockSpec((KW_ROWS, KW), lambda i: (i, 0)),
    ],
    out_specs=[
        pl.BlockSpec((KV_ROWS, D), lambda i: (i, 0)),
        pl.BlockSpec((KV_ROWS, D), lambda i: (i, 0)),
        pl.BlockSpec((KW_ROWS, KW), lambda i: (i, 0)),
    ],
    compiler_params=pltpu.CompilerParams(dimension_semantics=("parallel",)),
)


def _blend_kernel(pos_ref, k_val, v_val, kw_val, k_in, v_in, kw_in,
                  k_out, v_out, kw_out):
    sub = lax.broadcasted_iota(jnp.int32, (1, 1, GRP, 1), 2)
    hit = sub == pos_ref[0] % GRP
    k_out[...] = jnp.where(hit, k_val[...], k_in[...])
    v_out[...] = jnp.where(hit, v_val[...], v_in[...])
    kw_out[...] = jnp.where(hit, kw_val[...], kw_in[...])


_BLEND_SPEC = pltpu.PrefetchScalarGridSpec(
    num_scalar_prefetch=1,
    grid=(1,),
    in_specs=[
        pl.BlockSpec((BN, 1, 1, D), lambda i, pos: (0, 0, 0, 0)),
        pl.BlockSpec((BN, 1, 1, D), lambda i, pos: (0, 0, 0, 0)),
        pl.BlockSpec((B, 1, 1, KW), lambda i, pos: (0, 0, 0, 0)),
        pl.BlockSpec((BN, 1, GRP, D), lambda i, pos: (0, pos[0] // GRP, 0, 0)),
        pl.BlockSpec((BN, 1, GRP, D), lambda i, pos: (0, pos[0] // GRP, 0, 0)),
        pl.BlockSpec((B, 1, GRP, KW), lambda i, pos: (0, pos[0] // GRP, 0, 0)),
    ],
    out_specs=[
        pl.BlockSpec((BN, 1, GRP, D), lambda i, pos: (0, pos[0] // GRP, 0, 0)),
        pl.BlockSpec((BN, 1, GRP, D), lambda i, pos: (0, pos[0] // GRP, 0, 0)),
        pl.BlockSpec((B, 1, GRP, KW), lambda i, pos: (0, pos[0] // GRP, 0, 0)),
    ],
)


def kernel(input_pos, k_val, v_val, kw_val, k_cache, v_cache, kw_cache):
    pos = input_pos.astype(jnp.int32) % SEQ
    dt = k_cache.dtype
    bc = lambda x: lax.bitcast_convert_type(x, jnp.bfloat16)

    k_c, v_c, kw_c = pl.pallas_call(
        _copy_kernel,
        out_shape=[
            jax.ShapeDtypeStruct((BN * SEQ, D), jnp.bfloat16),
            jax.ShapeDtypeStruct((BN * SEQ, D), jnp.bfloat16),
            jax.ShapeDtypeStruct((B * SEQ, KW), jnp.bfloat16),
        ],
        **_COPY_SPEC,
    )(
        bc(k_cache.reshape(BN * SEQ, D)),
        bc(v_cache.reshape(BN * SEQ, D)),
        bc(kw_cache.reshape(B * SEQ, KW)),
    )
    k_out, v_out, kw_out = pl.pallas_call(
        _blend_kernel,
        grid_spec=_BLEND_SPEC,
        out_shape=[
            jax.ShapeDtypeStruct((BN, NG, GRP, D), jnp.bfloat16),
            jax.ShapeDtypeStruct((BN, NG, GRP, D), jnp.bfloat16),
            jax.ShapeDtypeStruct((B, NG, GRP, KW), jnp.bfloat16),
        ],
        input_output_aliases={4: 0, 5: 1, 6: 2},
    )(
        pos,
        bc(k_val.reshape(BN, 1, 1, D)),
        bc(v_val.reshape(BN, 1, 1, D)),
        bc(kw_val.reshape(B, 1, 1, KW)),
        k_c.reshape(BN, NG, GRP, D),
        v_c.reshape(BN, NG, GRP, D),
        kw_c.reshape(B, NG, GRP, KW),
    )
    return (
        lax.bitcast_convert_type(k_out.reshape(B, N, SEQ, D), dt),
        lax.bitcast_convert_type(v_out.reshape(B, N, SEQ, D), dt),
        lax.bitcast_convert_type(kw_out.reshape(B, SEQ, 2, N, N), dt),
    )


# speedup vs baseline: 1.0024x; 1.0024x over previous
import jax
import jax.numpy as jnp
from jax import lax
from jax.experimental import pallas as pl
from jax.experimental.pallas import tpu as pltpu

B = 16
N = 16
D = 128
SEQ = 2048
KW = 2 * N * N
BN = B * N
GRP = 16
NG = SEQ // GRP
G = 32
KV_ROWS = BN * SEQ // G
KW_ROWS = B * SEQ // G


def _copy_kernel(k_in, v_in, kw_in, k_out, v_out, kw_out):
    k_out[...] = k_in[...]
    v_out[...] = v_in[...]
    kw_out[...] = kw_in[...]


_COPY_SPEC = dict(
    grid=(G,),
    in_specs=[
        pl.BlockSpec((KV_ROWS, D), lambda i: (i, 0)),
        pl.BlockSpec((KV_ROWS, D), lambda i: (i, 0)),
        pl.BlockSpec((KW_ROWS, KW), lambda i: (i, 0)),
    ],
    out_specs=[
        pl.BlockSpec((KV_ROWS, D), lambda i: (i, 0)),
        pl.BlockSpec((KV_ROWS, D), lambda i: (i, 0)),
        pl.BlockSpec((KW_ROWS, KW), lambda i: (i, 0)),
    ],
    compiler_params=pltpu.CompilerParams(dimension_semantics=("parallel",)),
)


def _blend_kernel(pos_ref, k_val, v_val, kw_val, k_in, v_in, kw_in,
                  k_out, v_out, kw_out):
    sub = lax.broadcasted_iota(jnp.int32, (1, 1, GRP, 1), 2)
    hit = sub == pos_ref[0] % GRP
    k_out[...] = jnp.where(hit, k_val[...], k_in[...])
    v_out[...] = jnp.where(hit, v_val[...], v_in[...])
    kw_out[...] = jnp.where(hit, kw_val[...], kw_in[...])


_BLEND_SPEC = pltpu.PrefetchScalarGridSpec(
    num_scalar_prefetch=1,
    grid=(1,),
    in_specs=[
        pl.BlockSpec((BN, 1, 1, D), lambda i, pos: (0, 0, 0, 0)),
        pl.BlockSpec((BN, 1, 1, D), lambda i, pos: (0, 0, 0, 0)),
        pl.BlockSpec((B, 1, 1, KW), lambda i, pos: (0, 0, 0, 0)),
        pl.BlockSpec((BN, 1, GRP, D), lambda i, pos: (0, pos[0] // GRP, 0, 0)),
        pl.BlockSpec((BN, 1, GRP, D), lambda i, pos: (0, pos[0] // GRP, 0, 0)),
        pl.BlockSpec((B, 1, GRP, KW), lambda i, pos: (0, pos[0] // GRP, 0, 0)),
    ],
    out_specs=[
        pl.BlockSpec((BN, 1, GRP, D), lambda i, pos: (0, pos[0] // GRP, 0, 0)),
        pl.BlockSpec((BN, 1, GRP, D), lambda i, pos: (0, pos[0] // GRP, 0, 0)),
        pl.BlockSpec((B, 1, GRP, KW), lambda i, pos: (0, pos[0] // GRP, 0, 0)),
    ],
)


def kernel(input_pos, k_val, v_val, kw_val, k_cache, v_cache, kw_cache):
    pos = input_pos.astype(jnp.int32) % SEQ
    dt = k_cache.dtype
    bc = lambda x: lax.bitcast_convert_type(x, jnp.bfloat16)

    k_c, v_c, kw_c = pl.pallas_call(
        _copy_kernel,
        out_shape=[
            jax.ShapeDtypeStruct((BN * SEQ, D), jnp.bfloat16),
            jax.ShapeDtypeStruct((BN * SEQ, D), jnp.bfloat16),
            jax.ShapeDtypeStruct((B * SEQ, KW), jnp.bfloat16),
        ],
        **_COPY_SPEC,
    )(
        bc(k_cache.reshape(BN * SEQ, D)),
        bc(v_cache.reshape(BN * SEQ, D)),
        bc(kw_cache.reshape(B * SEQ, KW)),
    )
    k_out, v_out, kw_out = pl.pallas_call(
        _blend_kernel,
        grid_spec=_BLEND_SPEC,
        out_shape=[
            jax.ShapeDtypeStruct((BN, NG, GRP, D), jnp.bfloat16),
            jax.ShapeDtypeStruct((BN, NG, GRP, D), jnp.bfloat16),
            jax.ShapeDtypeStruct((B, NG, GRP, KW), jnp.bfloat16),
        ],
        input_output_aliases={4: 0, 5: 1, 6: 2},
    )(
        pos,
        bc(k_val.reshape(BN, 1, 1, D)),
        bc(v_val.reshape(BN, 1, 1, D)),
        bc(kw_val.reshape(B, 1, 1, KW)),
        k_c.reshape(BN, NG, GRP, D),
        v_c.reshape(BN, NG, GRP, D),
        kw_c.reshape(B, NG, GRP, KW),
    )
    return (
        lax.bitcast_convert_type(k_out.reshape(B, N, SEQ, D), dt),
        lax.bitcast_convert_type(v_out.reshape(B, N, SEQ, D), dt),
        lax.bitcast_convert_type(kw_out.reshape(B, SEQ, 2, N, N), dt),
    )
